# baseline (speedup 1.0000x reference)
import jax
import jax.numpy as jnp
from jax import lax
from jax.experimental import pallas as pl
from jax.experimental.pallas import tpu as pltpu
from jax.experimental.pallas import tpu_sc as plsc

B, N, T = 8, 5000, 50
LANES = 16
N_PAD = 5120
CHUNK = N_PAD // 4
T_PAD = 80
N_GROUPS = CHUNK // LANES
NEG_INF = float("-inf")
BIG = 2**30


def _sc_body(ps_s_hbm, ps_e_hbm, ts_s_hbm, ts_e_hbm,
             out_idx_hbm, out_val_hbm,
             ps_s_v, ps_e_v, ts_s_v, ts_e_v, oidx_v, oval_v,
             stage_v, mrg_v, sh_win, sem):
    c = lax.axis_index("c")
    s = lax.axis_index("s")
    bloc = s // 4
    chunk = s % 4
    b = c * 4 + bloc
    w = b * 4 + chunk
    base_n = chunk * CHUNK

    cp1 = pltpu.make_async_copy(ps_s_hbm.at[w], ps_s_v, sem)
    cp1.start()
    cp2 = pltpu.make_async_copy(ps_e_hbm.at[w], ps_e_v.at[pl.ds(0, CHUNK)], sem)
    cp2.start()
    cp3 = pltpu.make_async_copy(ts_s_hbm.at[b], ts_s_v, sem)
    cp3.start()
    cp4 = pltpu.make_async_copy(ts_e_hbm.at[b], ts_e_v, sem)
    cp4.start()
    cp1.wait()
    cp2.wait()
    cp3.wait()
    cp4.wait()

    lane_iota = lax.broadcasted_iota(jnp.int32, (LANES,), 0)
    minus_inf = jnp.float32(NEG_INF)

    def init_step(g, _):
        idxv = lane_iota + (base_n + g * LANES)
        pe = ps_e_v[pl.ds(g * LANES, LANES)]
        ps_e_v[pl.ds(g * LANES, LANES)] = jnp.where(
            idxv < N, pe, minus_inf)
        return 0

    lax.fori_loop(0, N_GROUPS, init_step, 0, unroll=8)

    def t_step(t, _):
        ts = ts_s_v[pl.ds(t, LANES)][0]
        te = ts_e_v[pl.ds(t, LANES)][0]
        lt = te - ts

        def g_step(g, carry):
            bv, bg = carry
            base = g * LANES
            ps = ps_s_v[pl.ds(base, LANES)]
            pe = ps_e_v[pl.ds(base, LANES)]
            inter = jnp.maximum(
                jnp.minimum(te, pe) - jnp.maximum(ts, ps), 0.0)
            lp = pe - ps
            union = lt + lp - inter
            iou = inter / jnp.maximum(union, 1e-8)
            enclose = jnp.maximum(te, pe) - jnp.minimum(ts, ps)
            score = iou - (enclose - union) / jnp.maximum(enclose, 1e-8)
            upd = score > bv
            bv = jnp.where(upd, score, bv)
            bg = jnp.where(upd, g, bg)
            return bv, bg

        bv, bg = lax.fori_loop(
            0, N_GROUPS, g_step,
            (jnp.full((LANES,), NEG_INF, jnp.float32),
             jnp.zeros((LANES,), jnp.int32)), unroll=16)
        bi = bg * LANES + lane_iota

        bvf = jnp.maximum(bv, lax.rev(bv, (0,)))
        m = bvf[0]
        for l in range(1, LANES // 2):
            m = jnp.maximum(m, bvf[l])
        cand = jnp.where(bv == m, bi, BIG)
        candf = jnp.minimum(cand, lax.rev(cand, (0,)))
        lidx = candf[0]
        for l in range(1, LANES // 2):
            lidx = jnp.minimum(lidx, candf[l])

        parity = t % 2
        stage_v[pl.ds(0, LANES)] = jnp.full((LANES,), m, jnp.float32)
        stage_v[pl.ds(LANES, LANES)] = jnp.full(
            (LANES,), (lidx + base_n).astype(jnp.float32), jnp.float32)
        slot = parity * (16 * 2 * LANES) + s * (2 * LANES)
        pltpu.sync_copy(stage_v, sh_win.at[pl.ds(slot, 2 * LANES)])
        plsc.subcore_barrier()

        roff = parity * (16 * 2 * LANES) + bloc * (4 * 2 * LANES)
        pltpu.sync_copy(sh_win.at[pl.ds(roff, 4 * 2 * LANES)], mrg_v)
        mvals = [mrg_v[pl.ds(k * 2 * LANES, LANES)][0] for k in range(4)]
        midxs = [mrg_v[pl.ds(k * 2 * LANES + LANES, LANES)][0]
                 .astype(jnp.int32) for k in range(4)]
        mg = mvals[0]
        for k in range(1, 4):
            mg = jnp.maximum(mg, mvals[k])
        gidx = jnp.int32(BIG)
        for k in range(4):
            gidx = jnp.where(mvals[k] == mg,
                             jnp.minimum(gidx, midxs[k]), gidx)

        loc = gidx - base_n

        @pl.when(jnp.logical_and(loc >= 0, loc < CHUNK))
        def _():
            lane0 = lane_iota == 0
            vu = ps_e_v[pl.ds(loc, LANES)]
            ps_e_v[pl.ds(loc, LANES)] = jnp.where(lane0, minus_inf, vu)

        @pl.when(chunk == 0)
        def _():
            lane0 = lane_iota == 0
            vi = oidx_v[pl.ds(t, LANES)]
            oidx_v[pl.ds(t, LANES)] = jnp.where(lane0, gidx, vi)
            vv = oval_v[pl.ds(t, LANES)]
            oval_v[pl.ds(t, LANES)] = jnp.where(lane0, mg, vv)

        return 0

    lax.fori_loop(0, T, t_step, 0)

    @pl.when(chunk == 0)
    def _():
        pltpu.sync_copy(oidx_v, out_idx_hbm.at[b])
        pltpu.sync_copy(oval_v, out_val_hbm.at[b])


@jax.jit
def kernel(pred_logits, pred_segments, tgt_segments, prediction_duration):
    del pred_logits
    scale = prediction_duration[:, None, None]
    ps = pred_segments * scale
    ts = tgt_segments * scale
    ps_s = jnp.pad(ps[..., 0], ((0, 0), (0, N_PAD - N))).reshape(B * 4, CHUNK)
    ps_e = jnp.pad(ps[..., 1], ((0, 0), (0, N_PAD - N))).reshape(B * 4, CHUNK)
    ts_s = jnp.pad(ts[..., 0], ((0, 0), (0, T_PAD - T)))
    ts_e = jnp.pad(ts[..., 1], ((0, 0), (0, T_PAD - T)))

    mesh = plsc.VectorSubcoreMesh(core_axis_name="c", subcore_axis_name="s")
    run = pl.kernel(
        _sc_body,
        out_type=(jax.ShapeDtypeStruct((B, T_PAD), jnp.int32),
                  jax.ShapeDtypeStruct((B, T_PAD), jnp.float32)),
        mesh=mesh,
        scratch_types=[
            pltpu.VMEM((CHUNK,), jnp.float32),
            pltpu.VMEM((CHUNK + LANES,), jnp.float32),
            pltpu.VMEM((T_PAD,), jnp.float32),
            pltpu.VMEM((T_PAD,), jnp.float32),
            pltpu.VMEM((T_PAD,), jnp.int32),
            pltpu.VMEM((T_PAD,), jnp.float32),
            pltpu.VMEM((2 * LANES,), jnp.float32),
            pltpu.VMEM((4 * 2 * LANES,), jnp.float32),
            pltpu.VMEM_SHARED((2 * 16 * 2 * LANES,), jnp.float32),
            pltpu.SemaphoreType.DMA,
        ],
    )
    out_idx, out_val = run(ps_s, ps_e, ts_s, ts_e)
    return (out_idx[:, :T].astype(jnp.int64),
            out_val[:, :T])


# speedup vs baseline: 1.1482x; 1.1482x over previous
import jax
import jax.numpy as jnp
from jax import lax
from jax.experimental import pallas as pl
from jax.experimental.pallas import tpu as pltpu
from jax.experimental.pallas import tpu_sc as plsc

B, N, T = 8, 5000, 50
LANES = 16
N_PAD = 5120
CHUNK = N_PAD // 4
T_PAD = 80
N_GROUPS = CHUNK // LANES
NEG_INF = float("-inf")
BIG = 2**30


def _sc_body(ps_s_hbm, ps_e_hbm, ts_s_hbm, ts_e_hbm,
             out_idx_hbm, out_val_hbm,
             ps_s_v, ps_e_v, ts_s_v, ts_e_v, oidx_v, oval_v,
             stage_v, mrg_v, sh_win, sem):
    c = lax.axis_index("c")
    s = lax.axis_index("s")
    bloc = s // 4
    chunk = s % 4
    b = c * 4 + bloc
    w = b * 4 + chunk
    base_n = chunk * CHUNK

    cp1 = pltpu.make_async_copy(ps_s_hbm.at[w], ps_s_v, sem)
    cp1.start()
    cp2 = pltpu.make_async_copy(ps_e_hbm.at[w], ps_e_v.at[pl.ds(0, CHUNK)], sem)
    cp2.start()
    cp3 = pltpu.make_async_copy(ts_s_hbm.at[b], ts_s_v, sem)
    cp3.start()
    cp4 = pltpu.make_async_copy(ts_e_hbm.at[b], ts_e_v, sem)
    cp4.start()
    cp1.wait()
    cp2.wait()
    cp3.wait()
    cp4.wait()

    lane_iota = lax.broadcasted_iota(jnp.int32, (LANES,), 0)
    minus_inf = jnp.float32(NEG_INF)

    def init_step(g, _):
        idxv = lane_iota + (base_n + g * LANES)
        pe = ps_e_v[pl.ds(g * LANES, LANES)]
        ps_e_v[pl.ds(g * LANES, LANES)] = jnp.where(
            idxv < N, pe, minus_inf)
        return 0

    lax.fori_loop(0, N_GROUPS, init_step, 0, unroll=8)

    def t_step(t, _):
        ts = ts_s_v[pl.ds(t, LANES)][0]
        te = ts_e_v[pl.ds(t, LANES)][0]
        lt = te - ts

        def g_step(g, carry):
            bv, bg = carry
            base = g * LANES
            ps = ps_s_v[pl.ds(base, LANES)]
            pe = ps_e_v[pl.ds(base, LANES)]
            inter = jnp.maximum(
                jnp.minimum(te, pe) - jnp.maximum(ts, ps), 0.0)
            lp = pe - ps
            union = lt + lp - inter
            iou = inter / jnp.maximum(union, 1e-8)
            enclose = jnp.maximum(te, pe) - jnp.minimum(ts, ps)
            score = iou - (enclose - union) / jnp.maximum(enclose, 1e-8)
            upd = score > bv
            bv = jnp.where(upd, score, bv)
            bg = jnp.where(upd, g, bg)
            return bv, bg

        bv, bg = lax.fori_loop(
            0, N_GROUPS, g_step,
            (jnp.full((LANES,), NEG_INF, jnp.float32),
             jnp.zeros((LANES,), jnp.int32)), unroll=8)
        bi = bg * LANES + lane_iota

        bvf = jnp.maximum(bv, lax.rev(bv, (0,)))
        m = bvf[0]
        for l in range(1, LANES // 2):
            m = jnp.maximum(m, bvf[l])
        cand = jnp.where(bv == m, bi, BIG)
        candf = jnp.minimum(cand, lax.rev(cand, (0,)))
        lidx = candf[0]
        for l in range(1, LANES // 2):
            lidx = jnp.minimum(lidx, candf[l])

        parity = t % 2
        stage_v[pl.ds(0, LANES)] = jnp.full((LANES,), m, jnp.float32)
        stage_v[pl.ds(LANES, LANES)] = jnp.full(
            (LANES,), (lidx + base_n).astype(jnp.float32), jnp.float32)
        slot = parity * (16 * 2 * LANES) + s * (2 * LANES)
        pltpu.sync_copy(stage_v, sh_win.at[pl.ds(slot, 2 * LANES)])
        plsc.subcore_barrier()

        roff = parity * (16 * 2 * LANES) + bloc * (4 * 2 * LANES)
        pltpu.sync_copy(sh_win.at[pl.ds(roff, 4 * 2 * LANES)], mrg_v)
        mvals = [mrg_v[pl.ds(k * 2 * LANES, LANES)][0] for k in range(4)]
        midxs = [mrg_v[pl.ds(k * 2 * LANES + LANES, LANES)][0]
                 .astype(jnp.int32) for k in range(4)]
        mg = mvals[0]
        for k in range(1, 4):
            mg = jnp.maximum(mg, mvals[k])
        gidx = jnp.int32(BIG)
        for k in range(4):
            gidx = jnp.where(mvals[k] == mg,
                             jnp.minimum(gidx, midxs[k]), gidx)

        loc = gidx - base_n

        @pl.when(jnp.logical_and(loc >= 0, loc < CHUNK))
        def _():
            lane0 = lane_iota == 0
            vu = ps_e_v[pl.ds(loc, LANES)]
            ps_e_v[pl.ds(loc, LANES)] = jnp.where(lane0, minus_inf, vu)

        @pl.when(chunk == 0)
        def _():
            lane0 = lane_iota == 0
            vi = oidx_v[pl.ds(t, LANES)]
            oidx_v[pl.ds(t, LANES)] = jnp.where(lane0, gidx, vi)
            vv = oval_v[pl.ds(t, LANES)]
            oval_v[pl.ds(t, LANES)] = jnp.where(lane0, mg, vv)

        return 0

    lax.fori_loop(0, T, t_step, 0)

    @pl.when(chunk == 0)
    def _():
        pltpu.sync_copy(oidx_v, out_idx_hbm.at[b])
        pltpu.sync_copy(oval_v, out_val_hbm.at[b])


@jax.jit
def kernel(pred_logits, pred_segments, tgt_segments, prediction_duration):
    del pred_logits
    scale = prediction_duration[:, None, None]
    ps = pred_segments * scale
    ts = tgt_segments * scale
    ps_s = jnp.pad(ps[..., 0], ((0, 0), (0, N_PAD - N))).reshape(B * 4, CHUNK)
    ps_e = jnp.pad(ps[..., 1], ((0, 0), (0, N_PAD - N))).reshape(B * 4, CHUNK)
    ts_s = jnp.pad(ts[..., 0], ((0, 0), (0, T_PAD - T)))
    ts_e = jnp.pad(ts[..., 1], ((0, 0), (0, T_PAD - T)))

    mesh = plsc.VectorSubcoreMesh(core_axis_name="c", subcore_axis_name="s")
    run = pl.kernel(
        _sc_body,
        out_type=(jax.ShapeDtypeStruct((B, T_PAD), jnp.int32),
                  jax.ShapeDtypeStruct((B, T_PAD), jnp.float32)),
        mesh=mesh,
        scratch_types=[
            pltpu.VMEM((CHUNK,), jnp.float32),
            pltpu.VMEM((CHUNK + LANES,), jnp.float32),
            pltpu.VMEM((T_PAD,), jnp.float32),
            pltpu.VMEM((T_PAD,), jnp.float32),
            pltpu.VMEM((T_PAD,), jnp.int32),
            pltpu.VMEM((T_PAD,), jnp.float32),
            pltpu.VMEM((2 * LANES,), jnp.float32),
            pltpu.VMEM((4 * 2 * LANES,), jnp.float32),
            pltpu.VMEM_SHARED((2 * 16 * 2 * LANES,), jnp.float32),
            pltpu.SemaphoreType.DMA,
        ],
    )
    out_idx, out_val = run(ps_s, ps_e, ts_s, ts_e)
    return (out_idx[:, :T].astype(jnp.int64),
            out_val[:, :T])
